# baseline (speedup 1.0000x reference)
import functools

import jax
import jax.numpy as jnp
from jax import lax
from jax.experimental import pallas as pl
from jax.experimental.pallas import tpu as pltpu
from jax.experimental.pallas import tpu_sc as plsc

N_NODES = 10000
N_EDGES = 320000
D_IN = 128
D_HID = 64
D_OUT = 128

NC = 2
NS = 16
NW = NC * NS

N_PAD = 10240
ROWS_PER_SUB = N_PAD // NS
CHUNK = 512
E_PAD = 327680
CPT = E_PAD // NW // CHUNK
BLK = 256
GRID = N_PAD // BLK

_mesh = plsc.VectorSubcoreMesh(
    core_axis_name="c", subcore_axis_name="s", num_cores=NC, num_subcores=NS
)
_sc_params = pltpu.CompilerParams(use_tc_tiling_on_sc=False)


def _sc_agg_body(u_hbm, zero_hbm, src_hbm, dst_hbm, out_hbm,
                 u_sh, acc_sh, sidx_v, didx_v, buf_v):
    c = lax.axis_index("c")
    s = lax.axis_index("s")
    wid = c * NS + s
    row0 = s * ROWS_PER_SUB
    pltpu.sync_copy(u_hbm.at[pl.ds(row0, ROWS_PER_SUB)],
                    u_sh.at[pl.ds(row0, ROWS_PER_SUB)])
    pltpu.sync_copy(zero_hbm.at[pl.ds(row0, ROWS_PER_SUB)],
                    acc_sh.at[pl.ds(row0, ROWS_PER_SUB)])
    plsc.subcore_barrier()

    def body(j, carry):
        pltpu.sync_copy(src_hbm.at[wid, j], sidx_v)
        pltpu.sync_copy(dst_hbm.at[wid, j], didx_v)
        pltpu.sync_copy(u_sh.at[sidx_v], buf_v)
        pltpu.sync_copy(buf_v, acc_sh.at[didx_v], add=True)
        return carry

    lax.fori_loop(0, CPT, body, 0)
    plsc.subcore_barrier()
    pltpu.sync_copy(acc_sh.at[pl.ds(row0, ROWS_PER_SUB)],
                    out_hbm.at[c, pl.ds(row0, ROWS_PER_SUB)])


_sc_agg = functools.partial(
    pl.kernel,
    out_type=jax.ShapeDtypeStruct((NC, N_PAD, D_HID), jnp.float32),
    mesh=_mesh,
    scratch_types=[
        pltpu.VMEM_SHARED((N_PAD, D_HID), jnp.float32),
        pltpu.VMEM_SHARED((N_PAD, D_HID), jnp.float32),
        pltpu.VMEM((CHUNK,), jnp.int32),
        pltpu.VMEM((CHUNK,), jnp.int32),
        pltpu.VMEM((CHUNK, D_HID), jnp.float32),
    ],
    compiler_params=_sc_params,
)(_sc_agg_body)


def _sc_deg_body(ones_hbm, zero_hbm, dst_hbm, out_hbm,
                 deg_sh, dst_v, ones_v):
    c = lax.axis_index("c")
    s = lax.axis_index("s")
    wid = c * NS + s
    row0 = s * ROWS_PER_SUB
    pltpu.sync_copy(zero_hbm.at[pl.ds(row0, ROWS_PER_SUB)],
                    deg_sh.at[pl.ds(row0, ROWS_PER_SUB)])
    pltpu.sync_copy(ones_hbm, ones_v)
    plsc.subcore_barrier()

    def body(j, carry):
        pltpu.sync_copy(dst_hbm.at[wid, j], dst_v)
        pltpu.sync_copy(ones_v, deg_sh.at[dst_v], add=True)
        return carry

    lax.fori_loop(0, CPT, body, 0)
    plsc.subcore_barrier()
    pltpu.sync_copy(deg_sh.at[pl.ds(row0, ROWS_PER_SUB)],
                    out_hbm.at[c, pl.ds(row0, ROWS_PER_SUB)])


_sc_deg = functools.partial(
    pl.kernel,
    out_type=jax.ShapeDtypeStruct((NC, N_PAD, 16), jnp.float32),
    mesh=_mesh,
    scratch_types=[
        pltpu.VMEM_SHARED((N_PAD, 16), jnp.float32),
        pltpu.VMEM((CHUNK,), jnp.int32),
        pltpu.VMEM((CHUNK, 16), jnp.float32),
    ],
    compiler_params=_sc_params,
)(_sc_deg_body)


def _tc_h0_body(x_ref, w_ref, h_ref):
    h_ref[:N_NODES] = jnp.dot(x_ref[...], w_ref[...],
                              preferred_element_type=jnp.float32)
    h_ref[N_NODES:] = jnp.zeros((N_PAD - N_NODES, D_HID), jnp.float32)


def _tc_h0(x, W0):
    return pl.pallas_call(
        _tc_h0_body,
        out_shape=jax.ShapeDtypeStruct((N_PAD, D_HID), jnp.float32),
    )(x, W0)


def _tc_scale_body(h_ref, degp_ref, u_ref, dis_ref):
    deg = degp_ref[0, :, :1] + degp_ref[1, :, :1] + 1.0
    dis = jnp.where(deg > 0, lax.rsqrt(deg), 0.0)
    u_ref[...] = dis * h_ref[...]
    dis_ref[...] = dis


def _tc_scale(h, degp):
    return pl.pallas_call(
        _tc_scale_body,
        out_shape=[
            jax.ShapeDtypeStruct((N_PAD, D_HID), jnp.float32),
            jax.ShapeDtypeStruct((N_PAD, 1), jnp.float32),
        ],
    )(h, degp)


def _ln_relu(t, g, be, eps=1e-5):
    mu = jnp.mean(t, axis=-1, keepdims=True)
    var = jnp.mean((t - mu) ** 2, axis=-1, keepdims=True)
    tn = (t - mu) * lax.rsqrt(var + eps) * g + be
    return jnp.maximum(tn, 0.0)


def _tc_mid_body(has_res, aggp_ref, u_ref, dis_ref, b_ref, g_ref,
                 be_ref, wn_ref, *rest):
    if has_res:
        hprev_ref, h_ref, un_ref = rest
    else:
        h_ref, un_ref = rest
    dis = dis_ref[...]
    t = dis * (aggp_ref[0] + aggp_ref[1] + u_ref[...]) + b_ref[...]
    h = _ln_relu(t, g_ref[...], be_ref[...])
    if has_res:
        h = hprev_ref[...] + h
    h_ref[...] = h
    un_ref[...] = dis * jnp.dot(h, wn_ref[...],
                                preferred_element_type=jnp.float32)


def _tc_mid(aggp, u, dis, b, g, be, Wn, hprev=None):
    has_res = hprev is not None
    ins = [aggp, u, dis, b.reshape(1, D_HID), g.reshape(1, D_HID),
           be.reshape(1, D_HID), Wn]
    if has_res:
        ins.append(hprev)
    return pl.pallas_call(
        functools.partial(_tc_mid_body, has_res),
        out_shape=[
            jax.ShapeDtypeStruct((N_PAD, D_HID), jnp.float32),
            jax.ShapeDtypeStruct((N_PAD, D_HID), jnp.float32),
        ],
    )(*ins)


def _tc_fin_body(aggp_ref, u_ref, dis_ref, b_ref, g_ref, be_ref,
                 hprev_ref, wa_ref, ba_ref, wb_ref, bb_ref, out_ref):
    dis = dis_ref[...]
    t = dis * (aggp_ref[0] + aggp_ref[1] + u_ref[...]) + b_ref[...]
    h = hprev_ref[...] + _ln_relu(t, g_ref[...], be_ref[...])
    row = lax.broadcasted_iota(jnp.int32, (N_PAD, 1), 0)
    h = jnp.where(row < N_NODES, h, 0.0)
    mean = jnp.sum(h, axis=0, keepdims=True) * (1.0 / N_NODES)
    hid = jnp.maximum(
        jnp.dot(mean, wa_ref[...], preferred_element_type=jnp.float32)
        + ba_ref[...], 0.0)
    out_ref[...] = (
        jnp.dot(hid, wb_ref[...], preferred_element_type=jnp.float32)
        + bb_ref[...])


def _tc_fin(aggp, u, dis, b, g, be, hprev, Wa, ba, Wb, bb):
    return pl.pallas_call(
        _tc_fin_body,
        out_shape=jax.ShapeDtypeStruct((1, D_OUT), jnp.float32),
    )(aggp, u, dis, b.reshape(1, D_HID), g.reshape(1, D_HID),
      be.reshape(1, D_HID), hprev, Wa, ba.reshape(1, D_HID), Wb,
      bb.reshape(1, D_OUT))


def kernel(x, edge_index, W0, b0, W1, b1, W2, b2, g0, be0, g1, be1, g2, be2,
           Wa, ba, Wb, bb):
    src = edge_index[0].astype(jnp.int32)
    dst = edge_index[1].astype(jnp.int32)
    pad = jnp.full((E_PAD - N_EDGES,), N_NODES, dtype=jnp.int32)
    src_r = jnp.concatenate([src, pad]).reshape(NW, CPT, CHUNK)
    dst_r = jnp.concatenate([dst, pad]).reshape(NW, CPT, CHUNK)

    zeros64 = jnp.zeros((N_PAD, D_HID), jnp.float32)
    zeros16 = jnp.zeros((N_PAD, 16), jnp.float32)
    ones16 = jnp.ones((CHUNK, 16), jnp.float32)

    h0 = _tc_h0(x, W0)
    degp = _sc_deg(ones16, zeros16, dst_r)
    u0, dis = _tc_scale(h0, degp)

    agg0 = _sc_agg(u0, zeros64, src_r, dst_r)
    h1, u1 = _tc_mid(agg0, u0, dis, b0, g0, be0, W1)

    agg1 = _sc_agg(u1, zeros64, src_r, dst_r)
    h2, u2 = _tc_mid(agg1, u1, dis, b1, g1, be1, W2, hprev=h1)

    agg2 = _sc_agg(u2, zeros64, src_r, dst_r)
    return _tc_fin(agg2, u2, dis, b2, g2, be2, h2, Wa, ba, Wb, bb)


# speedup vs baseline: 1.3524x; 1.3524x over previous
import functools

import jax
import jax.numpy as jnp
from jax import lax
from jax.experimental import pallas as pl
from jax.experimental.pallas import tpu as pltpu
from jax.experimental.pallas import tpu_sc as plsc

N_NODES = 10000
N_EDGES = 320000
D_IN = 128
D_HID = 64
D_OUT = 128

NC = 2
NS = 16
NW = NC * NS

N_PAD = 10240
ROWS_PER_SUB = N_PAD // NS
CHUNK = 1024
E_PAD = 327680
CPT = E_PAD // NW // CHUNK
BLK = 256
GRID = N_PAD // BLK

_mesh = plsc.VectorSubcoreMesh(
    core_axis_name="c", subcore_axis_name="s", num_cores=NC, num_subcores=NS
)
_sc_params = pltpu.CompilerParams(use_tc_tiling_on_sc=False)


def _sc_agg_body(u_hbm, zero_hbm, src_hbm, dst_hbm, out_hbm,
                 u_sh, acc_sh, sidx_v, didx_v, buf_v):
    c = lax.axis_index("c")
    s = lax.axis_index("s")
    wid = c * NS + s
    row0 = s * ROWS_PER_SUB
    pltpu.sync_copy(u_hbm.at[pl.ds(row0, ROWS_PER_SUB)],
                    u_sh.at[pl.ds(row0, ROWS_PER_SUB)])
    pltpu.sync_copy(zero_hbm.at[pl.ds(row0, ROWS_PER_SUB)],
                    acc_sh.at[pl.ds(row0, ROWS_PER_SUB)])
    plsc.subcore_barrier()

    def body(j, carry):
        pltpu.sync_copy(src_hbm.at[wid, j], sidx_v)
        pltpu.sync_copy(dst_hbm.at[wid, j], didx_v)
        pltpu.sync_copy(u_sh.at[sidx_v], buf_v)
        pltpu.sync_copy(buf_v, acc_sh.at[didx_v], add=True)
        return carry

    lax.fori_loop(0, CPT, body, 0)
    plsc.subcore_barrier()
    pltpu.sync_copy(acc_sh.at[pl.ds(row0, ROWS_PER_SUB)],
                    out_hbm.at[c, pl.ds(row0, ROWS_PER_SUB)])


_sc_agg = functools.partial(
    pl.kernel,
    out_type=jax.ShapeDtypeStruct((NC, N_PAD, D_HID), jnp.bfloat16),
    mesh=_mesh,
    scratch_types=[
        pltpu.VMEM_SHARED((N_PAD, D_HID), jnp.bfloat16),
        pltpu.VMEM_SHARED((N_PAD, D_HID), jnp.bfloat16),
        pltpu.VMEM((CHUNK,), jnp.int32),
        pltpu.VMEM((CHUNK,), jnp.int32),
        pltpu.VMEM((CHUNK, D_HID), jnp.bfloat16),
    ],
    compiler_params=_sc_params,
)(_sc_agg_body)


def _sc_deg_body(ones_hbm, zero_hbm, dst_hbm, out_hbm,
                 deg_sh, dst_v, ones_v):
    c = lax.axis_index("c")
    s = lax.axis_index("s")
    wid = c * NS + s
    row0 = s * ROWS_PER_SUB
    pltpu.sync_copy(zero_hbm.at[pl.ds(row0, ROWS_PER_SUB)],
                    deg_sh.at[pl.ds(row0, ROWS_PER_SUB)])
    pltpu.sync_copy(ones_hbm, ones_v)
    plsc.subcore_barrier()

    def body(j, carry):
        pltpu.sync_copy(dst_hbm.at[wid, j], dst_v)
        pltpu.sync_copy(ones_v, deg_sh.at[dst_v], add=True)
        return carry

    lax.fori_loop(0, CPT, body, 0)
    plsc.subcore_barrier()
    pltpu.sync_copy(deg_sh.at[pl.ds(row0, ROWS_PER_SUB)],
                    out_hbm.at[c, pl.ds(row0, ROWS_PER_SUB)])


_sc_deg = functools.partial(
    pl.kernel,
    out_type=jax.ShapeDtypeStruct((NC, N_PAD, 16), jnp.float32),
    mesh=_mesh,
    scratch_types=[
        pltpu.VMEM_SHARED((N_PAD, 16), jnp.float32),
        pltpu.VMEM((CHUNK,), jnp.int32),
        pltpu.VMEM((CHUNK, 16), jnp.float32),
    ],
    compiler_params=_sc_params,
)(_sc_deg_body)


def _tc_h0_body(x_ref, w_ref, h_ref):
    h_ref[:N_NODES] = jnp.dot(x_ref[...], w_ref[...],
                              preferred_element_type=jnp.float32)
    h_ref[N_NODES:] = jnp.zeros((N_PAD - N_NODES, D_HID), jnp.float32)


def _tc_h0(x, W0):
    return pl.pallas_call(
        _tc_h0_body,
        out_shape=jax.ShapeDtypeStruct((N_PAD, D_HID), jnp.float32),
    )(x, W0)


def _tc_scale_body(h_ref, degp_ref, u_ref, ub_ref, dis_ref):
    deg = degp_ref[0, :, :1] + degp_ref[1, :, :1] + 1.0
    dis = jnp.where(deg > 0, lax.rsqrt(deg), 0.0)
    u = dis * h_ref[...]
    u_ref[...] = u
    ub_ref[...] = u.astype(jnp.bfloat16)
    dis_ref[...] = dis


def _tc_scale(h, degp):
    return pl.pallas_call(
        _tc_scale_body,
        out_shape=[
            jax.ShapeDtypeStruct((N_PAD, D_HID), jnp.float32),
            jax.ShapeDtypeStruct((N_PAD, D_HID), jnp.bfloat16),
            jax.ShapeDtypeStruct((N_PAD, 1), jnp.float32),
        ],
    )(h, degp)


def _ln_relu(t, g, be, eps=1e-5):
    mu = jnp.mean(t, axis=-1, keepdims=True)
    var = jnp.mean((t - mu) ** 2, axis=-1, keepdims=True)
    tn = (t - mu) * lax.rsqrt(var + eps) * g + be
    return jnp.maximum(tn, 0.0)


def _tc_mid_body(has_res, aggp_ref, u_ref, dis_ref, b_ref, g_ref,
                 be_ref, wn_ref, *rest):
    if has_res:
        hprev_ref, h_ref, un_ref, unb_ref = rest
    else:
        h_ref, un_ref, unb_ref = rest
    dis = dis_ref[...]
    agg = (aggp_ref[0].astype(jnp.float32) + aggp_ref[1].astype(jnp.float32))
    t = dis * (agg + u_ref[...]) + b_ref[...]
    h = _ln_relu(t, g_ref[...], be_ref[...])
    if has_res:
        h = hprev_ref[...] + h
    h_ref[...] = h
    un = dis * jnp.dot(h, wn_ref[...], preferred_element_type=jnp.float32)
    un_ref[...] = un
    unb_ref[...] = un.astype(jnp.bfloat16)


def _tc_mid(aggp, u, dis, b, g, be, Wn, hprev=None):
    has_res = hprev is not None
    ins = [aggp, u, dis, b.reshape(1, D_HID), g.reshape(1, D_HID),
           be.reshape(1, D_HID), Wn]
    if has_res:
        ins.append(hprev)
    return pl.pallas_call(
        functools.partial(_tc_mid_body, has_res),
        out_shape=[
            jax.ShapeDtypeStruct((N_PAD, D_HID), jnp.float32),
            jax.ShapeDtypeStruct((N_PAD, D_HID), jnp.float32),
            jax.ShapeDtypeStruct((N_PAD, D_HID), jnp.bfloat16),
        ],
    )(*ins)


def _tc_fin_body(aggp_ref, u_ref, dis_ref, b_ref, g_ref, be_ref,
                 hprev_ref, wa_ref, ba_ref, wb_ref, bb_ref, out_ref):
    dis = dis_ref[...]
    agg = (aggp_ref[0].astype(jnp.float32) + aggp_ref[1].astype(jnp.float32))
    t = dis * (agg + u_ref[...]) + b_ref[...]
    h = hprev_ref[...] + _ln_relu(t, g_ref[...], be_ref[...])
    row = lax.broadcasted_iota(jnp.int32, (N_PAD, 1), 0)
    h = jnp.where(row < N_NODES, h, 0.0)
    mean = jnp.sum(h, axis=0, keepdims=True) * (1.0 / N_NODES)
    hid = jnp.maximum(
        jnp.dot(mean, wa_ref[...], preferred_element_type=jnp.float32)
        + ba_ref[...], 0.0)
    out_ref[...] = (
        jnp.dot(hid, wb_ref[...], preferred_element_type=jnp.float32)
        + bb_ref[...])


def _tc_fin(aggp, u, dis, b, g, be, hprev, Wa, ba, Wb, bb):
    return pl.pallas_call(
        _tc_fin_body,
        out_shape=jax.ShapeDtypeStruct((1, D_OUT), jnp.float32),
    )(aggp, u, dis, b.reshape(1, D_HID), g.reshape(1, D_HID),
      be.reshape(1, D_HID), hprev, Wa, ba.reshape(1, D_HID), Wb,
      bb.reshape(1, D_OUT))


def kernel(x, edge_index, W0, b0, W1, b1, W2, b2, g0, be0, g1, be1, g2, be2,
           Wa, ba, Wb, bb):
    src = edge_index[0].astype(jnp.int32)
    dst = edge_index[1].astype(jnp.int32)
    pad = jnp.full((E_PAD - N_EDGES,), N_NODES, dtype=jnp.int32)
    src_r = jnp.concatenate([src, pad]).reshape(NW, CPT, CHUNK)
    dst_r = jnp.concatenate([dst, pad]).reshape(NW, CPT, CHUNK)

    zeros64 = jnp.zeros((N_PAD, D_HID), jnp.bfloat16)
    zeros16 = jnp.zeros((N_PAD, 16), jnp.float32)
    ones16 = jnp.ones((CHUNK, 16), jnp.float32)

    h0 = _tc_h0(x, W0)
    degp = _sc_deg(ones16, zeros16, dst_r)
    u0, u0b, dis = _tc_scale(h0, degp)

    agg0 = _sc_agg(u0b, zeros64, src_r, dst_r)
    h1, u1, u1b = _tc_mid(agg0, u0, dis, b0, g0, be0, W1)

    agg1 = _sc_agg(u1b, zeros64, src_r, dst_r)
    h2, u2, u2b = _tc_mid(agg1, u1, dis, b1, g1, be1, W2, hprev=h1)

    agg2 = _sc_agg(u2b, zeros64, src_r, dst_r)
    return _tc_fin(agg2, u2, dis, b2, g2, be2, h2, Wa, ba, Wb, bb)
